# Initial kernel scaffold; baseline (speedup 1.0000x reference)
import jax
import jax.numpy as jnp
from jax.experimental import pallas as pl


def kernel(positions, sequence_ids, pos_table, seq_table):
    raise NotImplementedError("write your pallas kernel here")


# speedup vs baseline: 9.5775x; 9.5775x over previous
import functools

import jax
import jax.numpy as jnp
from jax import lax
from jax.experimental import pallas as pl
from jax.experimental.pallas import tpu as pltpu
from jax.experimental.pallas import tpu_sc as plsc

_LANES = 16
_GATHER = 128


@functools.lru_cache(maxsize=None)
def _build_sc_kernel(n_rows: int, n_ctx: int, d: int, chunk: int,
                     nc: int, ns: int):
    nw = nc * ns
    rows_per_w = n_rows // nw
    n_chunks = rows_per_w // chunk
    d_vecs = d // _LANES
    n_gathers = chunk // _GATHER

    mesh = plsc.VectorSubcoreMesh(core_axis_name="c", subcore_axis_name="s")

    @functools.partial(
        pl.kernel,
        out_type=jax.ShapeDtypeStruct((n_rows, d), jnp.float32),
        mesh=mesh,
        scratch_types=[
            pltpu.VMEM((n_ctx, d), jnp.float32),
            pltpu.VMEM((2, d), jnp.float32),
            pltpu.VMEM((2 * n_ctx, d), jnp.float32),
            pltpu.VMEM_SHARED((2 * n_ctx, d), jnp.float32),
            pltpu.VMEM((chunk,), jnp.int32),
            pltpu.VMEM((chunk,), jnp.int32),
            pltpu.VMEM((chunk,), jnp.int32),
            pltpu.VMEM((chunk, d), jnp.float32),
            pltpu.SemaphoreType.DMA,
        ],
        compiler_params=pltpu.CompilerParams(use_tc_tiling_on_sc=False),
    )
    def sc_kernel(pos_hbm, seq_hbm, ptab_hbm, stab_hbm, out_hbm,
                  ptab_v, stab_v, comb_v, comb_sh, posv, seqv, idxv, outv,
                  sem):
        c = lax.axis_index("c")
        s = lax.axis_index("s")
        wid = c * ns + s

        @pl.when(s == 0)
        def _build():
            pltpu.sync_copy(ptab_hbm, ptab_v)
            pltpu.sync_copy(stab_hbm, stab_v)

            def row(p, carry):
                for dc in range(d_vecs):
                    sl = pl.ds(dc * _LANES, _LANES)
                    v = ptab_v[p, sl]
                    comb_v[p, sl] = v + stab_v[0, sl]
                    comb_v[n_ctx + p, sl] = v + stab_v[1, sl]
                return carry

            lax.fori_loop(0, n_ctx, row, 0)
            pltpu.sync_copy(comb_v, comb_sh)

        plsc.subcore_barrier()

        def chunk_body(m, carry):
            base = wid * rows_per_w + m * chunk
            pltpu.sync_copy(pos_hbm.at[pl.ds(base, chunk)], posv)
            pltpu.sync_copy(seq_hbm.at[pl.ds(base, chunk)], seqv)

            def fuse(i, carry2):
                sl = pl.ds(i * _LANES, _LANES)
                idxv[sl] = seqv[sl] * n_ctx + posv[sl]
                return carry2

            lax.fori_loop(0, chunk // _LANES, fuse, 0)

            copies = []
            for g in range(n_gathers):
                gsl = pl.ds(g * _GATHER, _GATHER)
                copies.append(
                    pltpu.async_copy(comb_sh.at[idxv.at[gsl]],
                                     outv.at[gsl], sem))
            for cp in copies:
                cp.wait()

            pltpu.sync_copy(outv, out_hbm.at[pl.ds(base, chunk)])
            return carry

        lax.fori_loop(0, n_chunks, chunk_body, 0)

    return sc_kernel


def kernel(positions, sequence_ids, pos_table, seq_table):
    b, s = positions.shape
    n_ctx, d = pos_table.shape
    n_rows = b * s

    info = plsc.get_sparse_core_info()
    nc, ns = info.num_cores, info.num_subcores
    nw = nc * ns

    chunk = 1024
    assert n_rows % (nw * chunk) == 0 and d % _LANES == 0

    pos_flat = positions.reshape(n_rows).astype(jnp.int32)
    seq_flat = sequence_ids.reshape(n_rows).astype(jnp.int32)

    sc = _build_sc_kernel(n_rows, n_ctx, d, chunk, nc, ns)
    out = sc(pos_flat, seq_flat,
             pos_table.astype(jnp.float32), seq_table.astype(jnp.float32))
    return out.reshape(b, s, d)


# speedup vs baseline: 10.8202x; 1.1298x over previous
import functools

import jax
import jax.numpy as jnp
from jax import lax
from jax.experimental import pallas as pl
from jax.experimental.pallas import tpu as pltpu
from jax.experimental.pallas import tpu_sc as plsc

_LANES = 16
_GATHER = 128
_NBUF = 2


@functools.lru_cache(maxsize=None)
def _build_sc_kernel(n_rows: int, n_ctx: int, d: int, chunk: int,
                     nc: int, ns: int):
    nw = nc * ns
    rows_per_w = n_rows // nw
    n_chunks = rows_per_w // chunk
    d_vecs = d // _LANES
    n_gathers = chunk // _GATHER

    mesh = plsc.VectorSubcoreMesh(core_axis_name="c", subcore_axis_name="s")

    @functools.partial(
        pl.kernel,
        out_type=jax.ShapeDtypeStruct((n_rows, d), jnp.float32),
        mesh=mesh,
        scratch_types=[
            pltpu.VMEM((n_ctx, d), jnp.float32),
            pltpu.VMEM((2, d), jnp.float32),
            pltpu.VMEM((2 * n_ctx, d), jnp.float32),
            pltpu.VMEM_SHARED((2 * n_ctx, d), jnp.float32),
            [pltpu.VMEM((chunk,), jnp.int32)] * _NBUF,
            [pltpu.VMEM((chunk,), jnp.int32)] * _NBUF,
            [pltpu.VMEM((chunk,), jnp.int32)] * _NBUF,
            [pltpu.VMEM((chunk, d), jnp.float32)] * _NBUF,
            [pltpu.SemaphoreType.DMA] * _NBUF,
            [pltpu.SemaphoreType.DMA] * _NBUF,
            [pltpu.SemaphoreType.DMA] * _NBUF,
        ],
        compiler_params=pltpu.CompilerParams(use_tc_tiling_on_sc=False),
    )
    def sc_kernel(pos_hbm, seq_hbm, ptab_hbm, stab_hbm, out_hbm,
                  ptab_v, stab_v, comb_v, comb_sh, posv, seqv, idxv, outv,
                  sem_in, sem_g, sem_w):
        c = lax.axis_index("c")
        s = lax.axis_index("s")
        wid = c * ns + s
        w_base = wid * rows_per_w

        @pl.when(s == 0)
        def _build():
            pltpu.sync_copy(ptab_hbm, ptab_v)
            pltpu.sync_copy(stab_hbm, stab_v)

            def row(p, carry):
                for dc in range(d_vecs):
                    sl = pl.ds(dc * _LANES, _LANES)
                    v = ptab_v[p, sl]
                    comb_v[p, sl] = v + stab_v[0, sl]
                    comb_v[n_ctx + p, sl] = v + stab_v[1, sl]
                return carry

            lax.fori_loop(0, n_ctx, row, 0)
            pltpu.sync_copy(comb_v, comb_sh)

        plsc.subcore_barrier()

        def load_idx(m, b):
            base = w_base + m * chunk
            pltpu.async_copy(pos_hbm.at[pl.ds(base, chunk)], posv[b],
                             sem_in[b])
            pltpu.async_copy(seq_hbm.at[pl.ds(base, chunk)], seqv[b],
                             sem_in[b])

        def drain_idx(m, b):
            base = w_base + m * chunk
            pltpu.make_async_copy(pos_hbm.at[pl.ds(base, chunk)], posv[b],
                                  sem_in[b]).wait()
            pltpu.make_async_copy(seq_hbm.at[pl.ds(base, chunk)], seqv[b],
                                  sem_in[b]).wait()

        for b in range(_NBUF):
            load_idx(b, b)

        def super_body(m2, carry):
            for b in range(_NBUF):
                m = m2 * _NBUF + b
                drain_idx(m, b)

                def fuse(i, carry2):
                    sl = pl.ds(i * _LANES, _LANES)
                    idxv[b][sl] = seqv[b][sl] * n_ctx + posv[b][sl]
                    return carry2

                lax.fori_loop(0, chunk // _LANES, fuse, 0)

                @pl.when(m + _NBUF < n_chunks)
                def _prefetch():
                    load_idx(m + _NBUF, b)

                @pl.when(m2 > 0)
                def _drain_wb():
                    base_prev = w_base + (m - _NBUF) * chunk
                    pltpu.make_async_copy(
                        outv[b], out_hbm.at[pl.ds(base_prev, chunk)],
                        sem_w[b]).wait()

                copies = []
                for g in range(n_gathers):
                    gsl = pl.ds(g * _GATHER, _GATHER)
                    copies.append(
                        pltpu.async_copy(comb_sh.at[idxv[b].at[gsl]],
                                         outv[b].at[gsl], sem_g[b]))
                for cp in copies:
                    cp.wait()

                base = w_base + m * chunk
                pltpu.async_copy(outv[b], out_hbm.at[pl.ds(base, chunk)],
                                 sem_w[b])
            return carry

        lax.fori_loop(0, n_chunks // _NBUF, super_body, 0)

        for b in range(_NBUF):
            base = w_base + (n_chunks - _NBUF + b) * chunk
            pltpu.make_async_copy(outv[b], out_hbm.at[pl.ds(base, chunk)],
                                  sem_w[b]).wait()

    return sc_kernel


def kernel(positions, sequence_ids, pos_table, seq_table):
    b, s = positions.shape
    n_ctx, d = pos_table.shape
    n_rows = b * s

    info = plsc.get_sparse_core_info()
    nc, ns = info.num_cores, info.num_subcores
    nw = nc * ns

    chunk = 512
    assert n_rows % (nw * chunk * _NBUF) == 0 and d % _LANES == 0

    pos_flat = positions.reshape(n_rows).astype(jnp.int32)
    seq_flat = sequence_ids.reshape(n_rows).astype(jnp.int32)

    sc = _build_sc_kernel(n_rows, n_ctx, d, chunk, nc, ns)
    out = sc(pos_flat, seq_flat,
             pos_table.astype(jnp.float32), seq_table.astype(jnp.float32))
    return out.reshape(b, s, d)
